# baseline (speedup 1.0000x reference)
import jax
import jax.numpy as jnp
from jax.experimental import pallas as pl
from jax.experimental.pallas import tpu as pltpu

N = 16384
S = 128
FEA = 1024
HID = 128
D = 64
NC = 2

KB = 128
NKB = FEA // KB
CH = 2048
NCH = N // CH
CONV_PREC = jax.lax.Precision.DEFAULT
POST_PREC = jax.lax.Precision.DEFAULT


def _dot(a, b, prec):
    return jax.lax.dot_general(a, b, (((1,), (0,)), ((), ())),
                               preferred_element_type=jnp.float32,
                               precision=prec)


def _padded_slab(x_ref, base, length):
    lo = max(base, 0)
    hi = min(base + length, N)
    core = x_ref[lo:hi]
    parts = []
    if lo - base:
        parts.append(jnp.zeros((lo - base, core.shape[1]), core.dtype))
    parts.append(core)
    if base + length - hi:
        parts.append(jnp.zeros((base + length - hi, core.shape[1]),
                               core.dtype))
    return parts[0] if len(parts) == 1 else jnp.concatenate(parts, axis=0)


def _conv_body(x_ref, w_ref, y_ref):
    k = pl.program_id(0)
    w = w_ref[...]
    for c in range(NCH):
        r0 = c * CH
        base = r0 - S - 1
        xpad = _padded_slab(x_ref, base, CH + 2 * S + 2)
        le = CH + 2 * S
        x_c = xpad[1:1 + le]
        col = (jax.lax.broadcasted_iota(jnp.int32, (le, 1), 0)
               + (r0 - S)) & (S - 1)
        xl = jnp.where(col != 0, xpad[0:le], 0)
        xr = jnp.where(col != (S - 1), xpad[2:2 + le], 0)
        xcat = jnp.concatenate([xl, x_c, xr], axis=1)
        contrib = (_dot(xcat[S:S + CH], w[1, 0], CONV_PREC)
                   + _dot(xcat[:CH], w[0, 0], CONV_PREC)
                   + _dot(xcat[2 * S:], w[2, 0], CONV_PREC))

        @pl.when(k == 0)
        def _():
            y_ref[r0:r0 + CH, :] = contrib

        @pl.when(k > 0)
        def _():
            y_ref[r0:r0 + CH, :] += contrib


def _post_body(y_ref, g_ref, be_ref, wa_ref, ba_ref, wb_ref, bb_ref,
               wc_ref, bc_ref, wcl_ref, bcl_ref,
               fs_ref, yp_ref, yh_ref):
    tot = jnp.zeros((8, HID), jnp.float32)
    tot2 = jnp.zeros((8, HID), jnp.float32)
    for c in range(NCH):
        y = y_ref[c * CH:(c + 1) * CH]
        tot = tot + jnp.sum(y.reshape(CH // 8, 8, HID), axis=0)
        tot2 = tot2 + jnp.sum((y * y).reshape(CH // 8, 8, HID), axis=0)
    mean = jnp.sum(tot, axis=0, keepdims=True) / N
    var = jnp.sum(tot2, axis=0, keepdims=True) / N - mean * mean
    inv = g_ref[...] * jax.lax.rsqrt(var + 1e-5)
    shift = be_ref[...] - mean * inv

    dets = []
    clss = []
    for c in range(NCH):
        y = y_ref[c * CH:(c + 1) * CH]
        hh = jax.nn.relu(y * inv + shift)
        a = jnp.tanh(_dot(hh, wa_ref[...], POST_PREC) + ba_ref[...])
        b = jax.nn.sigmoid(_dot(hh, wb_ref[...], POST_PREC) + bb_ref[...])
        dets.append(_dot(a * b, wc_ref[...], POST_PREC) + bc_ref[...])
        clss.append(_dot(hh, wcl_ref[...], POST_PREC) + bcl_ref[...])
    det = jnp.concatenate(dets, axis=0)
    cls = jnp.concatenate(clss, axis=0)

    cm = jnp.max(cls, axis=1, keepdims=True)
    ce = jnp.exp(cls - cm)
    cls_score = ce / jnp.sum(ce, axis=1, keepdims=True)
    dm = jnp.max(det, axis=0, keepdims=True)
    de = jnp.exp(det - dm)
    det_score = de / jnp.sum(de, axis=0, keepdims=True)
    fs = cls_score * det_score
    fs_ref[...] = fs
    yp = jnp.clip(jnp.sum(fs, axis=0, keepdims=True), 1e-10, 1.0 - 1e-10)
    yp_ref[...] = yp
    yh_ref[...] = (yp[:, 1:2] > yp[:, 0:1]).astype(jnp.int32)


def kernel(h, conv_w, bn_gamma, bn_beta, Wa, ba, Wb, bb, Wc, bc, Wcls, bcls):
    wt = conv_w.transpose(2, 3, 1, 0)
    wr = wt.reshape(3, 3, NKB, KB, HID).transpose(0, 2, 1, 3, 4)
    wr = wr.reshape(3, NKB, 3 * KB, HID)

    y = pl.pallas_call(
        _conv_body,
        grid=(NKB,),
        in_specs=[
            pl.BlockSpec((N, KB), lambda k: (0, k)),
            pl.BlockSpec((3, 1, 3 * KB, HID), lambda k: (0, k, 0, 0)),
        ],
        out_specs=pl.BlockSpec((N, HID), lambda k: (0, 0)),
        out_shape=jax.ShapeDtypeStruct((N, HID), jnp.float32),
        compiler_params=pltpu.CompilerParams(
            dimension_semantics=("arbitrary",)),
    )(h.astype(jnp.bfloat16), wr.astype(jnp.bfloat16))

    fs, yp, yh = pl.pallas_call(
        _post_body,
        out_shape=(
            jax.ShapeDtypeStruct((N, NC), jnp.float32),
            jax.ShapeDtypeStruct((1, NC), jnp.float32),
            jax.ShapeDtypeStruct((1, 1), jnp.int32),
        ),
    )(y, bn_gamma.reshape(1, HID), bn_beta.reshape(1, HID),
      Wa.T, ba.reshape(1, D), Wb.T, bb.reshape(1, D),
      Wc.T, bc.reshape(1, NC), Wcls.T, bcls.reshape(1, NC))

    return (fs, yp.reshape(NC), yh.reshape(1))


# speedup vs baseline: 1.2647x; 1.2647x over previous
import jax
import jax.numpy as jnp
from jax.experimental import pallas as pl
from jax.experimental.pallas import tpu as pltpu

N = 16384
S = 128
FEA = 1024
HID = 128
D = 64
NC = 2

KB = 128
NKB = FEA // KB
CH = 2048
NCH = N // CH
CONV_PREC = jax.lax.Precision.DEFAULT
POST_PREC = jax.lax.Precision.DEFAULT


def _dot(a, b, prec):
    return jax.lax.dot_general(a, b, (((1,), (0,)), ((), ())),
                               preferred_element_type=jnp.float32,
                               precision=prec)


def _padded_slab(x_ref, base, length):
    lo = max(base, 0)
    hi = min(base + length, N)
    core = x_ref[lo:hi]
    parts = []
    if lo - base:
        parts.append(jnp.zeros((lo - base, core.shape[1]), core.dtype))
    parts.append(core)
    if base + length - hi:
        parts.append(jnp.zeros((base + length - hi, core.shape[1]),
                               core.dtype))
    return parts[0] if len(parts) == 1 else jnp.concatenate(parts, axis=0)


def _conv_body(x_ref, w_ref, y_ref):
    k = pl.program_id(0)
    w3 = w_ref[0]
    for c in range(NCH):
        r0 = c * CH
        base = r0 - S - 1
        xpad = _padded_slab(x_ref, base, CH + 2 * S + 2)
        le = CH + 2 * S
        x_c = xpad[1:1 + le]
        col = (jax.lax.broadcasted_iota(jnp.int32, (le, 1), 0)
               + (r0 - S)) & (S - 1)
        xl = jnp.where(col != 0, xpad[0:le], 0)
        xr = jnp.where(col != (S - 1), xpad[2:2 + le], 0)
        xcat = jnp.concatenate([xl, x_c, xr], axis=1)
        z = _dot(xcat, w3, CONV_PREC)
        contrib = (z[:CH, :HID]
                   + z[S:S + CH, HID:2 * HID]
                   + z[2 * S:, 2 * HID:])

        @pl.when(k == 0)
        def _():
            y_ref[r0:r0 + CH, :] = contrib

        @pl.when(k > 0)
        def _():
            y_ref[r0:r0 + CH, :] += contrib


def _post_body(y_ref, g_ref, be_ref, wa_ref, ba_ref, wb_ref, bb_ref,
               wc_ref, bc_ref, wcl_ref, bcl_ref,
               fs_ref, yp_ref, yh_ref):
    tot = jnp.zeros((8, HID), jnp.float32)
    tot2 = jnp.zeros((8, HID), jnp.float32)
    for c in range(NCH):
        y = y_ref[c * CH:(c + 1) * CH]
        tot = tot + jnp.sum(y.reshape(CH // 8, 8, HID), axis=0)
        tot2 = tot2 + jnp.sum((y * y).reshape(CH // 8, 8, HID), axis=0)
    mean = jnp.sum(tot, axis=0, keepdims=True) / N
    var = jnp.sum(tot2, axis=0, keepdims=True) / N - mean * mean
    inv = g_ref[...] * jax.lax.rsqrt(var + 1e-5)
    shift = be_ref[...] - mean * inv

    dets = []
    clss = []
    for c in range(NCH):
        y = y_ref[c * CH:(c + 1) * CH]
        hh = jax.nn.relu(y * inv + shift)
        a = jnp.tanh(_dot(hh, wa_ref[...], POST_PREC) + ba_ref[...])
        b = jax.nn.sigmoid(_dot(hh, wb_ref[...], POST_PREC) + bb_ref[...])
        dets.append(_dot(a * b, wc_ref[...], POST_PREC) + bc_ref[...])
        clss.append(_dot(hh, wcl_ref[...], POST_PREC) + bcl_ref[...])
    det = jnp.concatenate(dets, axis=0)
    cls = jnp.concatenate(clss, axis=0)

    cm = jnp.max(cls, axis=1, keepdims=True)
    ce = jnp.exp(cls - cm)
    cls_score = ce / jnp.sum(ce, axis=1, keepdims=True)
    dm = jnp.max(det, axis=0, keepdims=True)
    de = jnp.exp(det - dm)
    det_score = de / jnp.sum(de, axis=0, keepdims=True)
    fs = cls_score * det_score
    fs_ref[...] = fs
    yp = jnp.clip(jnp.sum(fs, axis=0, keepdims=True), 1e-10, 1.0 - 1e-10)
    yp_ref[...] = yp
    yh_ref[...] = (yp[:, 1:2] > yp[:, 0:1]).astype(jnp.int32)


def kernel(h, conv_w, bn_gamma, bn_beta, Wa, ba, Wb, bb, Wc, bc, Wcls, bcls):
    wt = conv_w.transpose(2, 3, 1, 0)
    wr = wt.reshape(3, 3, NKB, KB, HID).transpose(0, 2, 1, 3, 4)
    wr = wr.reshape(3, NKB, 3 * KB, HID)
    wcat = jnp.concatenate([wr[0], wr[1], wr[2]], axis=2)

    y = pl.pallas_call(
        _conv_body,
        grid=(NKB,),
        in_specs=[
            pl.BlockSpec((N, KB), lambda k: (0, k)),
            pl.BlockSpec((1, 3 * KB, 3 * HID), lambda k: (k, 0, 0)),
        ],
        out_specs=pl.BlockSpec((N, HID), lambda k: (0, 0)),
        out_shape=jax.ShapeDtypeStruct((N, HID), jnp.float32),
        compiler_params=pltpu.CompilerParams(
            dimension_semantics=("arbitrary",)),
    )(h, wcat)

    fs, yp, yh = pl.pallas_call(
        _post_body,
        out_shape=(
            jax.ShapeDtypeStruct((N, NC), jnp.float32),
            jax.ShapeDtypeStruct((1, NC), jnp.float32),
            jax.ShapeDtypeStruct((1, 1), jnp.int32),
        ),
    )(y, bn_gamma.reshape(1, HID), bn_beta.reshape(1, HID),
      Wa.T, ba.reshape(1, D), Wb.T, bb.reshape(1, D),
      Wc.T, bc.reshape(1, NC), Wcls.T, bcls.reshape(1, NC))

    return (fs, yp.reshape(NC), yh.reshape(1))


# speedup vs baseline: 1.3048x; 1.0317x over previous
import jax
import jax.numpy as jnp
from jax.experimental import pallas as pl
from jax.experimental.pallas import tpu as pltpu

N = 16384
S = 128
FEA = 1024
HID = 128
D = 64
NC = 2

KB = 128
NKB = FEA // KB
CH = 2048
NCH = N // CH
CONV_PREC = jax.lax.Precision.DEFAULT
POST_PREC = jax.lax.Precision.DEFAULT


def _dot(a, b, prec):
    return jax.lax.dot_general(a, b, (((1,), (0,)), ((), ())),
                               preferred_element_type=jnp.float32,
                               precision=prec)


def _padded_slab(x_ref, base, length):
    lo = max(base, 0)
    hi = min(base + length, N)
    core = x_ref[lo:hi]
    parts = []
    if lo - base:
        parts.append(jnp.zeros((lo - base, core.shape[1]), core.dtype))
    parts.append(core)
    if base + length - hi:
        parts.append(jnp.zeros((base + length - hi, core.shape[1]),
                               core.dtype))
    return parts[0] if len(parts) == 1 else jnp.concatenate(parts, axis=0)


def _conv_body(x_ref, w_ref, y_ref):
    k = pl.program_id(0)
    w3 = w_ref[0]
    for c in range(NCH):
        r0 = c * CH
        base = r0 - S - 1
        xpad = _padded_slab(x_ref, base, CH + 2 * S + 2)
        le = CH + 2 * S
        x_c = xpad[1:1 + le]
        col = (jax.lax.broadcasted_iota(jnp.int32, (le, 1), 0)
               + (r0 - S)) & (S - 1)
        xl = jnp.where(col != 0, xpad[0:le], 0)
        xr = jnp.where(col != (S - 1), xpad[2:2 + le], 0)
        xcat = jnp.concatenate([xl, x_c, xr], axis=1)
        z = _dot(xcat.astype(jnp.bfloat16), w3, CONV_PREC)
        contrib = (z[:CH, :HID]
                   + z[S:S + CH, HID:2 * HID]
                   + z[2 * S:, 2 * HID:])

        @pl.when(k == 0)
        def _():
            y_ref[r0:r0 + CH, :] = contrib

        @pl.when(k > 0)
        def _():
            y_ref[r0:r0 + CH, :] += contrib


def _post_body(y_ref, g_ref, be_ref, wa_ref, ba_ref, wb_ref, bb_ref,
               wc_ref, bc_ref, wcl_ref, bcl_ref,
               fs_ref, yp_ref, yh_ref):
    tot = jnp.zeros((8, HID), jnp.float32)
    tot2 = jnp.zeros((8, HID), jnp.float32)
    for c in range(NCH):
        y = y_ref[c * CH:(c + 1) * CH]
        tot = tot + jnp.sum(y.reshape(CH // 8, 8, HID), axis=0)
        tot2 = tot2 + jnp.sum((y * y).reshape(CH // 8, 8, HID), axis=0)
    mean = jnp.sum(tot, axis=0, keepdims=True) / N
    var = jnp.sum(tot2, axis=0, keepdims=True) / N - mean * mean
    inv = g_ref[...] * jax.lax.rsqrt(var + 1e-5)
    shift = be_ref[...] - mean * inv

    dets = []
    clss = []
    for c in range(NCH):
        y = y_ref[c * CH:(c + 1) * CH]
        hh = jax.nn.relu(y * inv + shift)
        a = jnp.tanh(_dot(hh, wa_ref[...], POST_PREC) + ba_ref[...])
        b = jax.nn.sigmoid(_dot(hh, wb_ref[...], POST_PREC) + bb_ref[...])
        dets.append(_dot(a * b, wc_ref[...], POST_PREC) + bc_ref[...])
        clss.append(_dot(hh, wcl_ref[...], POST_PREC) + bcl_ref[...])
    det = jnp.concatenate(dets, axis=0)
    cls = jnp.concatenate(clss, axis=0)

    cm = jnp.max(cls, axis=1, keepdims=True)
    ce = jnp.exp(cls - cm)
    cls_score = ce / jnp.sum(ce, axis=1, keepdims=True)
    dm = jnp.max(det, axis=0, keepdims=True)
    de = jnp.exp(det - dm)
    det_score = de / jnp.sum(de, axis=0, keepdims=True)
    fs = cls_score * det_score
    fs_ref[...] = fs
    yp = jnp.clip(jnp.sum(fs, axis=0, keepdims=True), 1e-10, 1.0 - 1e-10)
    yp_ref[...] = yp
    yh_ref[...] = (yp[:, 1:2] > yp[:, 0:1]).astype(jnp.int32)


def kernel(h, conv_w, bn_gamma, bn_beta, Wa, ba, Wb, bb, Wc, bc, Wcls, bcls):
    wt = conv_w.transpose(2, 3, 1, 0)
    wr = wt.reshape(3, 3, NKB, KB, HID).transpose(0, 2, 1, 3, 4)
    wr = wr.reshape(3, NKB, 3 * KB, HID)
    wcat = jnp.concatenate([wr[0], wr[1], wr[2]], axis=2)

    y = pl.pallas_call(
        _conv_body,
        grid=(NKB,),
        in_specs=[
            pl.BlockSpec((N, KB), lambda k: (0, k)),
            pl.BlockSpec((1, 3 * KB, 3 * HID), lambda k: (k, 0, 0)),
        ],
        out_specs=pl.BlockSpec((N, HID), lambda k: (0, 0)),
        out_shape=jax.ShapeDtypeStruct((N, HID), jnp.float32),
        compiler_params=pltpu.CompilerParams(
            dimension_semantics=("arbitrary",)),
    )(h, wcat.astype(jnp.bfloat16))

    fs, yp, yh = pl.pallas_call(
        _post_body,
        out_shape=(
            jax.ShapeDtypeStruct((N, NC), jnp.float32),
            jax.ShapeDtypeStruct((1, NC), jnp.float32),
            jax.ShapeDtypeStruct((1, 1), jnp.int32),
        ),
    )(y, bn_gamma.reshape(1, HID), bn_beta.reshape(1, HID),
      Wa.T, ba.reshape(1, D), Wb.T, bb.reshape(1, D),
      Wc.T, bc.reshape(1, NC), Wcls.T, bcls.reshape(1, NC))

    return (fs, yp.reshape(NC), yh.reshape(1))


# speedup vs baseline: 1.3591x; 1.0416x over previous
import jax
import jax.numpy as jnp
from jax.experimental import pallas as pl
from jax.experimental.pallas import tpu as pltpu

N = 16384
S = 128
FEA = 1024
HID = 128
D = 64
NC = 2

KB = 128
NKB = FEA // KB
CH = 2048
NCH = N // CH
PREC = jax.lax.Precision.DEFAULT


def _dot(a, b):
    return jax.lax.dot_general(a, b, (((1,), (0,)), ((), ())),
                               preferred_element_type=jnp.float32,
                               precision=PREC)


def _padded_slab(x_ref, base, length):
    lo = max(base, 0)
    hi = min(base + length, N)
    core = x_ref[lo:hi]
    parts = []
    if lo - base:
        parts.append(jnp.zeros((lo - base, core.shape[1]), core.dtype))
    parts.append(core)
    if base + length - hi:
        parts.append(jnp.zeros((base + length - hi, core.shape[1]),
                               core.dtype))
    return parts[0] if len(parts) == 1 else jnp.concatenate(parts, axis=0)


def _body(x_ref, w_ref, g_ref, be_ref, wa_ref, ba_ref, wb_ref, bb_ref,
          wc_ref, bc_ref, wcl_ref, bcl_ref,
          fs_ref, yp_ref, yh_ref, y_ref):
    k = pl.program_id(0)
    w3 = w_ref[0]
    for c in range(NCH):
        r0 = c * CH
        base = r0 - S - 1
        xpad = _padded_slab(x_ref, base, CH + 2 * S + 2)
        le = CH + 2 * S
        x_c = xpad[1:1 + le]
        col = (jax.lax.broadcasted_iota(jnp.int32, (le, 1), 0)
               + (r0 - S)) & (S - 1)
        xl = jnp.where(col != 0, xpad[0:le], 0)
        xr = jnp.where(col != (S - 1), xpad[2:2 + le], 0)
        xcat = jnp.concatenate([xl, x_c, xr], axis=1)
        z = _dot(xcat.astype(jnp.bfloat16), w3)
        contrib = (z[:CH, :HID]
                   + z[S:S + CH, HID:2 * HID]
                   + z[2 * S:, 2 * HID:])

        @pl.when(k == 0)
        def _():
            y_ref[r0:r0 + CH, :] = contrib

        @pl.when(k > 0)
        def _():
            y_ref[r0:r0 + CH, :] += contrib

    @pl.when(k == NKB - 1)
    def _post():
        tot = jnp.zeros((8, HID), jnp.float32)
        tot2 = jnp.zeros((8, HID), jnp.float32)
        for c in range(NCH):
            y = y_ref[c * CH:(c + 1) * CH]
            tot = tot + jnp.sum(y.reshape(CH // 8, 8, HID), axis=0)
            tot2 = tot2 + jnp.sum((y * y).reshape(CH // 8, 8, HID), axis=0)
        mean = jnp.sum(tot, axis=0, keepdims=True) / N
        var = jnp.sum(tot2, axis=0, keepdims=True) / N - mean * mean
        inv = g_ref[...] * jax.lax.rsqrt(var + 1e-5)
        shift = be_ref[...] - mean * inv

        dets = []
        clss = []
        for c in range(NCH):
            y = y_ref[c * CH:(c + 1) * CH]
            hh = jax.nn.relu(y * inv + shift)
            a = jnp.tanh(_dot(hh, wa_ref[...]) + ba_ref[...])
            b = jax.nn.sigmoid(_dot(hh, wb_ref[...]) + bb_ref[...])
            dets.append(_dot(a * b, wc_ref[...]) + bc_ref[...])
            clss.append(_dot(hh, wcl_ref[...]) + bcl_ref[...])
        det = jnp.concatenate(dets, axis=0)
        cls = jnp.concatenate(clss, axis=0)

        cm = jnp.max(cls, axis=1, keepdims=True)
        ce = jnp.exp(cls - cm)
        cls_score = ce / jnp.sum(ce, axis=1, keepdims=True)
        dm = jnp.max(det, axis=0, keepdims=True)
        de = jnp.exp(det - dm)
        det_score = de / jnp.sum(de, axis=0, keepdims=True)
        fs = cls_score * det_score
        fs_ref[...] = fs
        yp = jnp.clip(jnp.sum(fs, axis=0, keepdims=True), 1e-10, 1.0 - 1e-10)
        yp_ref[...] = yp
        yh_ref[...] = (yp[:, 1:2] > yp[:, 0:1]).astype(jnp.int32)


def kernel(h, conv_w, bn_gamma, bn_beta, Wa, ba, Wb, bb, Wc, bc, Wcls, bcls):
    wt = conv_w.transpose(2, 3, 1, 0)
    wr = wt.reshape(3, 3, NKB, KB, HID).transpose(0, 2, 1, 3, 4)
    wr = wr.reshape(3, NKB, 3 * KB, HID)
    wcat = jnp.concatenate([wr[0], wr[1], wr[2]], axis=2)

    def cmap(nd):
        return (lambda k: (0,) * nd)

    fs, yp, yh = pl.pallas_call(
        _body,
        grid=(NKB,),
        in_specs=[
            pl.BlockSpec((N, KB), lambda k: (0, k)),
            pl.BlockSpec((1, 3 * KB, 3 * HID), lambda k: (k, 0, 0)),
            pl.BlockSpec((1, HID), cmap(2)),
            pl.BlockSpec((1, HID), cmap(2)),
            pl.BlockSpec((HID, D), cmap(2)),
            pl.BlockSpec((1, D), cmap(2)),
            pl.BlockSpec((HID, D), cmap(2)),
            pl.BlockSpec((1, D), cmap(2)),
            pl.BlockSpec((D, NC), cmap(2)),
            pl.BlockSpec((1, NC), cmap(2)),
            pl.BlockSpec((HID, NC), cmap(2)),
            pl.BlockSpec((1, NC), cmap(2)),
        ],
        out_specs=(
            pl.BlockSpec((N, NC), cmap(2)),
            pl.BlockSpec((1, NC), cmap(2)),
            pl.BlockSpec((1, 1), cmap(2)),
        ),
        out_shape=(
            jax.ShapeDtypeStruct((N, NC), jnp.float32),
            jax.ShapeDtypeStruct((1, NC), jnp.float32),
            jax.ShapeDtypeStruct((1, 1), jnp.int32),
        ),
        scratch_shapes=[pltpu.VMEM((N, HID), jnp.float32)],
        compiler_params=pltpu.CompilerParams(
            dimension_semantics=("arbitrary",)),
    )(h, wcat.astype(jnp.bfloat16),
      bn_gamma.reshape(1, HID), bn_beta.reshape(1, HID),
      Wa.T, ba.reshape(1, D), Wb.T, bb.reshape(1, D),
      Wc.T, bc.reshape(1, NC), Wcls.T, bcls.reshape(1, NC))

    return (fs, yp.reshape(NC), yh.reshape(1))


# speedup vs baseline: 1.3777x; 1.0137x over previous
import jax
import jax.numpy as jnp
from jax.experimental import pallas as pl
from jax.experimental.pallas import tpu as pltpu

N = 16384
S = 128
FEA = 1024
HID = 128
D = 64
NC = 2

KB = 128
NKB = FEA // KB
CH = 2048
NCH = N // CH
PREC = jax.lax.Precision.DEFAULT


def _dot(a, b):
    return jax.lax.dot_general(a, b, (((1,), (0,)), ((), ())),
                               preferred_element_type=jnp.float32,
                               precision=PREC)


def _padded_slab(x_ref, base, length):
    lo = max(base, 0)
    hi = min(base + length, N)
    core = x_ref[lo:hi]
    parts = []
    if lo - base:
        parts.append(jnp.zeros((lo - base, core.shape[1]), core.dtype))
    parts.append(core)
    if base + length - hi:
        parts.append(jnp.zeros((base + length - hi, core.shape[1]),
                               core.dtype))
    return parts[0] if len(parts) == 1 else jnp.concatenate(parts, axis=0)


def _body(x_ref, w_ref, g_ref, be_ref, wab_ref, bab_ref,
          wc_ref, bc_ref, wcl_ref, bcl_ref,
          fs_ref, yp_ref, yh_ref, y_ref):
    k = pl.program_id(0)
    w3 = w_ref[0]
    for c in range(NCH):
        r0 = c * CH
        base = r0 - S - 1
        xpad = _padded_slab(x_ref, base, CH + 2 * S + 2)
        le = CH + 2 * S
        x_c = xpad[1:1 + le]
        col = (jax.lax.broadcasted_iota(jnp.int32, (le, 1), 0)
               + (r0 - S)) & (S - 1)
        xl = jnp.where(col != 0, xpad[0:le], 0)
        xr = jnp.where(col != (S - 1), xpad[2:2 + le], 0)
        xcat = jnp.concatenate([xl, x_c, xr], axis=1)
        z = _dot(xcat.astype(jnp.bfloat16), w3)
        contrib = (z[:CH, :HID]
                   + z[S:S + CH, HID:2 * HID]
                   + z[2 * S:, 2 * HID:])

        @pl.when(k == 0)
        def _():
            y_ref[r0:r0 + CH, :] = contrib

        @pl.when(k > 0)
        def _():
            y_ref[r0:r0 + CH, :] += contrib

    @pl.when(k == NKB - 1)
    def _post():
        tot = jnp.zeros((8, HID), jnp.float32)
        tot2 = jnp.zeros((8, HID), jnp.float32)
        for c in range(NCH):
            y = y_ref[c * CH:(c + 1) * CH]
            tot = tot + jnp.sum(y.reshape(CH // 8, 8, HID), axis=0)
            tot2 = tot2 + jnp.sum((y * y).reshape(CH // 8, 8, HID), axis=0)
        mean = jnp.sum(tot, axis=0, keepdims=True) / N
        var = jnp.sum(tot2, axis=0, keepdims=True) / N - mean * mean
        inv = g_ref[...] * jax.lax.rsqrt(var + 1e-5)
        shift = be_ref[...] - mean * inv

        dets = []
        clss = []
        for c in range(NCH):
            y = y_ref[c * CH:(c + 1) * CH]
            hh = jax.nn.relu(y * inv + shift)
            ab2 = _dot(hh, wab_ref[...]) + bab_ref[...]
            a = jnp.tanh(ab2[:, :D])
            b = jax.nn.sigmoid(ab2[:, D:])
            dets.append(_dot(a * b, wc_ref[...]) + bc_ref[...])
            clss.append(_dot(hh, wcl_ref[...]) + bcl_ref[...])
        det = jnp.concatenate(dets, axis=0)
        cls = jnp.concatenate(clss, axis=0)

        cm = jnp.max(cls, axis=1, keepdims=True)
        ce = jnp.exp(cls - cm)
        cls_score = ce / jnp.sum(ce, axis=1, keepdims=True)
        dm = jnp.max(det, axis=0, keepdims=True)
        de = jnp.exp(det - dm)
        det_score = de / jnp.sum(de, axis=0, keepdims=True)
        fs = cls_score * det_score
        fs_ref[...] = fs
        yp = jnp.clip(jnp.sum(fs, axis=0, keepdims=True), 1e-10, 1.0 - 1e-10)
        yp_ref[...] = yp
        yh_ref[...] = (yp[:, 1:2] > yp[:, 0:1]).astype(jnp.int32)


def kernel(h, conv_w, bn_gamma, bn_beta, Wa, ba, Wb, bb, Wc, bc, Wcls, bcls):
    wt = conv_w.transpose(2, 3, 1, 0)
    wr = wt.reshape(3, 3, NKB, KB, HID).transpose(0, 2, 1, 3, 4)
    wr = wr.reshape(3, NKB, 3 * KB, HID)
    wcat = jnp.concatenate([wr[0], wr[1], wr[2]], axis=2)

    def cmap(nd):
        return (lambda k: (0,) * nd)

    fs, yp, yh = pl.pallas_call(
        _body,
        grid=(NKB,),
        in_specs=[
            pl.BlockSpec((N, KB), lambda k: (0, k)),
            pl.BlockSpec((1, 3 * KB, 3 * HID), lambda k: (k, 0, 0)),
            pl.BlockSpec((1, HID), cmap(2)),
            pl.BlockSpec((1, HID), cmap(2)),
            pl.BlockSpec((HID, 2 * D), cmap(2)),
            pl.BlockSpec((1, 2 * D), cmap(2)),
            pl.BlockSpec((D, NC), cmap(2)),
            pl.BlockSpec((1, NC), cmap(2)),
            pl.BlockSpec((HID, NC), cmap(2)),
            pl.BlockSpec((1, NC), cmap(2)),
        ],
        out_specs=(
            pl.BlockSpec((N, NC), cmap(2)),
            pl.BlockSpec((1, NC), cmap(2)),
            pl.BlockSpec((1, 1), cmap(2)),
        ),
        out_shape=(
            jax.ShapeDtypeStruct((N, NC), jnp.float32),
            jax.ShapeDtypeStruct((1, NC), jnp.float32),
            jax.ShapeDtypeStruct((1, 1), jnp.int32),
        ),
        scratch_shapes=[pltpu.VMEM((N, HID), jnp.float32)],
        compiler_params=pltpu.CompilerParams(
            dimension_semantics=("arbitrary",)),
    )(h, wcat.astype(jnp.bfloat16),
      bn_gamma.reshape(1, HID), bn_beta.reshape(1, HID),
      jnp.concatenate([Wa.T, Wb.T], axis=1),
      jnp.concatenate([ba, bb]).reshape(1, 2 * D),
      Wc.T, bc.reshape(1, NC), Wcls.T, bcls.reshape(1, NC))

    return (fs, yp.reshape(NC), yh.reshape(1))
